# baseline (speedup 1.0000x reference)
import functools

import jax
import jax.numpy as jnp
from jax.experimental import pallas as pl
from jax.experimental.pallas import tpu as pltpu

_F32 = jnp.float32
_BF16 = jnp.bfloat16


def _stage1_body(x_ref, w0_ref, b0_ref, w1_ref, b1_ref, o_ref, s0_ref, *,
                 c_in, d, hw, wf):
    m_plane = hw - 2 * wf - 2
    hw_s = hw + 128

    ch0 = 2048
    n_chunks0 = hw // ch0

    omax = 2 * wf + 2

    def l0_chunk(dd, loc, cw):
        acc = b0_ref[...] + jnp.zeros((8, cw), _F32)
        for kd in range(3):
            xw = x_ref[:, pl.ds((dd + kd) * hw + loc, cw + omax)]
            res = jnp.dot(w0_ref[pl.ds(kd * 72, 72)], xw,
                          preferred_element_type=_F32)
            for r in range(9):
                kh, kw = divmod(r, 3)
                blk = jax.lax.slice_in_dim(res, r * 8, r * 8 + 8, axis=0)
                acc = acc + jax.lax.slice_in_dim(
                    blk, kh * wf + kw, kh * wf + kw + cw, axis=1)
        acc = jnp.maximum(acc, 0.0).astype(_F32)
        s0_ref[pl.ds(dd * 8, 8), pl.ds(loc, cw)] = acc

    def body0(k, carry):
        l0_chunk(k // n_chunks0, (k % n_chunks0) * ch0, ch0)
        return carry

    jax.lax.fori_loop(0, (d - 2) * n_chunks0 - 1, body0, 0)
    l0_chunk(d - 3, (n_chunks0 - 1) * ch0,
             m_plane - (n_chunks0 - 1) * ch0)

    ch1 = 896
    n_chunks1 = 18

    def l1_pair(p, c0):
        outs = []
        for half in range(2):
            dd = 2 * p + half
            xw = s0_ref[pl.ds(dd * 8, 24), pl.ds(c0, ch1 + omax)]
            res = jnp.dot(w1_ref[...], xw,
                          preferred_element_type=_F32)
            acc = b1_ref[...] + jnp.zeros((16, ch1), _F32)
            for r in range(9):
                kh, kw = divmod(r, 3)
                blk = jax.lax.slice_in_dim(res, r * 16, r * 16 + 16, axis=0)
                acc = acc + jax.lax.slice_in_dim(
                    blk, kh * wf + kw, kh * wf + kw + ch1, axis=1)
            outs.append(acc)
        mx = jnp.maximum(jnp.maximum(outs[0], outs[1]), 0.0).astype(_F32)
        o_ref[pl.ds(p * 16, 16), pl.ds(c0, ch1)] = mx

    def body1(j, carry):
        l1_pair(j // n_chunks1, (j % n_chunks1) * ch1)
        return carry

    jax.lax.fori_loop(0, 6 * n_chunks1, body1, 0)


def _pair_body(x_ref, wa_ref, ba_ref, wb_ref, bb_ref, o_ref, s_ref, *,
               cina, couta, coutb, fw, wf, chunks, relu_b):
    for dd in range(6):
        for c0, cw in chunks:
            acc = ba_ref[...] + jnp.zeros((couta, cw), _F32)
            for t in range(9):
                kh, kw = divmod(t, 3)
                acc = acc + jnp.dot(
                    wa_ref[t],
                    x_ref[pl.ds(dd * cina, 3 * cina),
                          pl.ds(kh * wf + kw + c0, cw)],
                    preferred_element_type=_F32)
            acc = jnp.maximum(acc, 0.0).astype(_F32)
            s_ref[pl.ds((dd + 1) * couta, couta), pl.ds(c0, cw)] = acc
    z = jnp.zeros((couta, fw), _F32)
    s_ref[pl.ds(0, couta), :] = z
    s_ref[pl.ds(7 * couta, couta), :] = z
    for dd in range(6):
        for c0, cw in chunks:
            acc = bb_ref[...] + jnp.zeros((coutb, cw), _F32)
            for t in range(9):
                kh, kw = divmod(t, 3)
                acc = acc + jnp.dot(
                    wb_ref[t],
                    s_ref[pl.ds(dd * couta, 3 * couta),
                          pl.ds(kh * wf + kw + c0, cw)],
                    preferred_element_type=_F32)
            if relu_b:
                acc = jnp.maximum(acc, 0.0)
            o_ref[pl.ds(dd * coutb, coutb), pl.ds(c0, cw)] = acc.astype(_F32)


def _w27(w):
    return jnp.transpose(w, (2, 3, 4, 0, 1)).reshape(27, w.shape[0], -1)


def _w9(w):
    cout, cin = w.shape[0], w.shape[1]
    return jnp.transpose(w, (3, 4, 0, 2, 1)).reshape(
        9, cout, 3 * cin).astype(_F32)


def _stage1(x, w0, b0, w1, b1):
    n, c, d, h, w = x.shape
    hw = h * w
    xf = x.reshape(n, c, d * hw)
    body = functools.partial(_stage1_body, c_in=c, d=d, hw=hw, wf=w)
    return pl.pallas_call(
        body,
        out_shape=jax.ShapeDtypeStruct((n, 96, hw), _F32),
        grid=(n,),
        in_specs=[
            pl.BlockSpec((None, c, d * hw), lambda i: (i, 0, 0)),
            pl.BlockSpec((216, c), lambda i: (0, 0)),
            pl.BlockSpec((8, 1), lambda i: (0, 0)),
            pl.BlockSpec((144, 24), lambda i: (0, 0)),
            pl.BlockSpec((16, 1), lambda i: (0, 0)),
        ],
        out_specs=pl.BlockSpec((None, 96, hw), lambda i: (i, 0, 0)),
        scratch_shapes=[pltpu.VMEM((112, hw + 128), _F32)],
        compiler_params=pltpu.CompilerParams(
            dimension_semantics=("parallel",),
            vmem_limit_bytes=60 * 1024 * 1024),
    )(xf, _w27(w0).reshape(216, c), b0.reshape(8, 1),
      _w9(w1).reshape(144, 24), b1.reshape(16, 1))


def _pair(x, wa, ba, wb, bb, *, cina, couta, coutb, fw, wf, chunks, relu_b):
    n = x.shape[0]
    body = functools.partial(
        _pair_body, cina=cina, couta=couta, coutb=coutb, fw=fw, wf=wf,
        chunks=chunks, relu_b=relu_b)
    return pl.pallas_call(
        body,
        out_shape=jax.ShapeDtypeStruct((n, 6 * coutb, fw), _F32),
        grid=(n,),
        in_specs=[
            pl.BlockSpec((None, 8 * cina, fw), lambda i: (i, 0, 0)),
            pl.BlockSpec((9, couta, 3 * cina), lambda i: (0, 0, 0)),
            pl.BlockSpec((couta, 1), lambda i: (0, 0)),
            pl.BlockSpec((9, coutb, 3 * couta), lambda i: (0, 0, 0)),
            pl.BlockSpec((coutb, 1), lambda i: (0, 0)),
        ],
        out_specs=pl.BlockSpec((None, 6 * coutb, fw), lambda i: (i, 0, 0)),
        scratch_shapes=[pltpu.VMEM((8 * couta, fw), _F32)],
        compiler_params=pltpu.CompilerParams(
            dimension_semantics=("parallel",),
            vmem_limit_bytes=60 * 1024 * 1024),
    )(x, _w9(wa), ba.reshape(couta, 1).astype(_F32),
      _w9(wb), bb.reshape(coutb, 1).astype(_F32))


def _tower(x5, params):
    n, c, d, h, w = x5.shape
    (w0, b0), (w1, b1), (w2, b2), (w3, b3), (w4, b4), (w5, b5), (w6, b6), \
        (w7, b7) = params

    y = _stage1(x5, w0, b0, w1, b1)

    y = y.reshape(n, 6, 16, h, w)[:, :, :, :h - 4, :w - 4]
    y = y.reshape(n, 6, 16, (h - 4) // 2, 2, (w - 4) // 2, 2).max(axis=(4, 6))
    h2, w2_ = (h - 4) // 2, (w - 4) // 2
    y = y.reshape(n, 6, 16, h2 * w2_)
    y = jnp.pad(y, ((0, 0), (1, 1), (0, 0), (0, 3968 - h2 * w2_)))
    y = y.reshape(n, 128, 3968)

    y = _pair(y, w2, b2, w3, b3, cina=16, couta=16, coutb=32, fw=3968,
              wf=w2_, chunks=((0, 1280), (1280, 1280), (2560, 1280)),
              relu_b=True)

    y = y.reshape(n, 6, 32, 3968)[:, :, :, :h2 * w2_]
    y = y.reshape(n, 6, 32, h2, w2_)[:, :, :, :h2 - 4, :w2_ - 4]
    y = y.reshape(n, 6, 32, (h2 - 4) // 2, 2, (w2_ - 4) // 2, 2)
    y = y.max(axis=(4, 6))
    h3, w3_ = (h2 - 4) // 2, (w2_ - 4) // 2
    y = y.reshape(n, 6, 32, h3 * w3_)
    y = jnp.pad(y, ((0, 0), (1, 1), (0, 0), (0, 896 - h3 * w3_)))
    y = y.reshape(n, 256, 896)

    y = _pair(y, w4, b4, w5, b5, cina=32, couta=64, coutb=32, fw=896,
              wf=w3_, chunks=((0, 448), (448, 333)), relu_b=True)

    y = y.reshape(n, 6, 32, 896)[:, :, :, :h3 * w3_]
    y = y.reshape(n, 6, 32, h3, w3_)[:, :, :, :(h3 - 4) // 2 * 2,
                                     :(w3_ - 4) // 2 * 2]
    h4, w4_ = (h3 - 4) // 2, (w3_ - 4) // 2
    y = y.reshape(n, 6, 32, h4, 2, w4_, 2).max(axis=(4, 6))
    y = y.reshape(n, 6, 32, h4 * w4_)
    y = jnp.pad(y, ((0, 0), (1, 1), (0, 0), (0, 256 - h4 * w4_)))
    y = y.reshape(n, 256, 256)

    y = _pair(y, w6, b6, w7, b7, cina=32, couta=64, coutb=32, fw=256,
              wf=w4_, chunks=((0, 118),), relu_b=False)

    y = y.reshape(n, 6, 32, 256)[:, :, :, :h4 * w4_]
    y = y.reshape(n, 6, 32, h4, w4_)[:, :, :, :h4 - 4, :w4_ - 4]
    y = jnp.transpose(y, (0, 1, 3, 4, 2))
    return y.reshape(n, 6 * (h4 - 4) * (w4_ - 4), 32)


def _head_body(a_ref, t_ref, wa_ref, wt_ref, b_ref, soft_ref, out_ref):
    sa = jnp.mean(a_ref[...], axis=1)
    st = jnp.mean(t_ref[...], axis=1)
    logits = (jnp.sum(sa * wa_ref[...], axis=1, keepdims=True)
              + jnp.sum(st * wt_ref[...], axis=1, keepdims=True)
              + b_ref[...])
    out_ref[...] = logits
    soft_ref[...] = jnp.ones_like(logits)


def kernel(dwi_x, t2_x,
           dwi_w0, dwi_b0, dwi_w1, dwi_b1, dwi_w2, dwi_b2, dwi_w3, dwi_b3,
           dwi_w4, dwi_b4, dwi_w5, dwi_b5, dwi_w6, dwi_b6, dwi_w7, dwi_b7,
           t2_w0, t2_b0, t2_w1, t2_b1, t2_w2, t2_b2, t2_w3, t2_b3,
           t2_w4, t2_b4, t2_w5, t2_b5, t2_w6, t2_b6, t2_w7, t2_b7,
           lin_w, lin_b):
    p_dwi = [(dwi_w0, dwi_b0), (dwi_w1, dwi_b1), (dwi_w2, dwi_b2),
             (dwi_w3, dwi_b3), (dwi_w4, dwi_b4), (dwi_w5, dwi_b5),
             (dwi_w6, dwi_b6), (dwi_w7, dwi_b7)]
    p_t2 = [(t2_w0, t2_b0), (t2_w1, t2_b1), (t2_w2, t2_b2), (t2_w3, t2_b3),
            (t2_w4, t2_b4), (t2_w5, t2_b5), (t2_w6, t2_b6), (t2_w7, t2_b7)]
    f_dwi = _tower(dwi_x, p_dwi)
    f_t2 = _tower(t2_x, p_t2)
    n = f_dwi.shape[0]
    soft, out = pl.pallas_call(
        _head_body,
        out_shape=(jax.ShapeDtypeStruct((n, 1), _F32),
                   jax.ShapeDtypeStruct((n, 1), _F32)),
    )(f_dwi, f_t2, lin_w[:, :32], lin_w[:, 32:], lin_b.reshape(1, 1))
    return soft, out


# speedup vs baseline: 1.2717x; 1.2717x over previous
import functools

import jax
import jax.numpy as jnp
from jax.experimental import pallas as pl
from jax.experimental.pallas import tpu as pltpu

_F32 = jnp.float32
_BF16 = jnp.bfloat16


def _stage1_body(x_ref, w0_ref, b0_ref, w1_ref, b1_ref, o_ref, s0_ref, *,
                 c_in, d, hw, wf):
    m_plane = hw - 2 * wf - 2
    hw_s = hw + 128

    ch0 = 4096
    n_chunks0 = hw // ch0

    omax = 2 * wf + 2

    def l0_chunk(dd, loc, cw):
        acc = b0_ref[...] + jnp.zeros((8, cw), _F32)
        for kd in range(3):
            xw = x_ref[:, pl.ds((dd + kd) * hw + loc, cw + omax)]
            for r in range(9):
                kh, kw = divmod(r, 3)
                sl = jax.lax.slice_in_dim(xw, kh * wf + kw,
                                          kh * wf + kw + cw, axis=1)
                acc = acc + jnp.dot(
                    w0_ref[pl.ds((kd * 9 + r) * 8, 8)], sl,
                    preferred_element_type=_F32)
        acc = jnp.maximum(acc, 0.0).astype(_F32)
        s0_ref[pl.ds(dd * 8, 8), pl.ds(loc, cw)] = acc

    def body0(k, carry):
        l0_chunk(k // n_chunks0, (k % n_chunks0) * ch0, ch0)
        return carry

    jax.lax.fori_loop(0, (d - 2) * n_chunks0 - 1, body0, 0)
    l0_chunk(d - 3, (n_chunks0 - 1) * ch0, m_plane - (n_chunks0 - 1) * ch0)

    ch1 = 1792
    n_chunks1 = 9

    def l1_pair(p, c0):
        outs = []
        for half in range(2):
            dd = 2 * p + half
            acc = b1_ref[...] + jnp.zeros((16, ch1), _F32)
            xw = s0_ref[pl.ds(dd * 8, 24), pl.ds(c0, ch1 + omax)]
            for t in range(9):
                kh, kw = divmod(t, 3)
                sl = jax.lax.slice_in_dim(xw, kh * wf + kw,
                                          kh * wf + kw + ch1, axis=1)
                acc = acc + jnp.dot(
                    w1_ref[pl.ds(t * 16, 16)], sl,
                    preferred_element_type=_F32)
            outs.append(acc)
        mx = jnp.maximum(jnp.maximum(outs[0], outs[1]), 0.0).astype(_F32)
        o_ref[pl.ds(p * 16, 16), pl.ds(c0, ch1)] = mx

    def body1(j, carry):
        l1_pair(j // n_chunks1, (j % n_chunks1) * ch1)
        return carry

    jax.lax.fori_loop(0, 6 * n_chunks1, body1, 0)


def _pair_body(x_ref, wa_ref, ba_ref, wb_ref, bb_ref, o_ref, s_ref, *,
               cina, couta, coutb, fw, wf, chunks, relu_b):
    for dd in range(6):
        for c0, cw in chunks:
            acc = ba_ref[...] + jnp.zeros((couta, cw), _F32)
            for t in range(9):
                kh, kw = divmod(t, 3)
                acc = acc + jnp.dot(
                    wa_ref[t],
                    x_ref[pl.ds(dd * cina, 3 * cina),
                          pl.ds(kh * wf + kw + c0, cw)],
                    preferred_element_type=_F32)
            acc = jnp.maximum(acc, 0.0).astype(_F32)
            s_ref[pl.ds((dd + 1) * couta, couta), pl.ds(c0, cw)] = acc
    z = jnp.zeros((couta, fw), _F32)
    s_ref[pl.ds(0, couta), :] = z
    s_ref[pl.ds(7 * couta, couta), :] = z
    for dd in range(6):
        for c0, cw in chunks:
            acc = bb_ref[...] + jnp.zeros((coutb, cw), _F32)
            for t in range(9):
                kh, kw = divmod(t, 3)
                acc = acc + jnp.dot(
                    wb_ref[t],
                    s_ref[pl.ds(dd * couta, 3 * couta),
                          pl.ds(kh * wf + kw + c0, cw)],
                    preferred_element_type=_F32)
            if relu_b:
                acc = jnp.maximum(acc, 0.0)
            o_ref[pl.ds(dd * coutb, coutb), pl.ds(c0, cw)] = acc.astype(_F32)


def _w27(w):
    return jnp.transpose(w, (2, 3, 4, 0, 1)).reshape(27, w.shape[0], -1)


def _w9(w):
    cout, cin = w.shape[0], w.shape[1]
    return jnp.transpose(w, (3, 4, 0, 2, 1)).reshape(
        9, cout, 3 * cin).astype(_F32)


def _stage1(x, w0, b0, w1, b1):
    n, c, d, h, w = x.shape
    hw = h * w
    xf = x.reshape(n, c, d * hw)
    body = functools.partial(_stage1_body, c_in=c, d=d, hw=hw, wf=w)
    return pl.pallas_call(
        body,
        out_shape=jax.ShapeDtypeStruct((n, 96, hw), _F32),
        grid=(n,),
        in_specs=[
            pl.BlockSpec((None, c, d * hw), lambda i: (i, 0, 0)),
            pl.BlockSpec((216, c), lambda i: (0, 0)),
            pl.BlockSpec((8, 1), lambda i: (0, 0)),
            pl.BlockSpec((144, 24), lambda i: (0, 0)),
            pl.BlockSpec((16, 1), lambda i: (0, 0)),
        ],
        out_specs=pl.BlockSpec((None, 96, hw), lambda i: (i, 0, 0)),
        scratch_shapes=[pltpu.VMEM((112, hw + 128), _F32)],
        compiler_params=pltpu.CompilerParams(
            dimension_semantics=("parallel",),
            vmem_limit_bytes=60 * 1024 * 1024),
    )(xf, _w27(w0).reshape(216, c), b0.reshape(8, 1),
      _w9(w1).reshape(144, 24), b1.reshape(16, 1))


def _pair(x, wa, ba, wb, bb, *, cina, couta, coutb, fw, wf, chunks, relu_b):
    n = x.shape[0]
    body = functools.partial(
        _pair_body, cina=cina, couta=couta, coutb=coutb, fw=fw, wf=wf,
        chunks=chunks, relu_b=relu_b)
    return pl.pallas_call(
        body,
        out_shape=jax.ShapeDtypeStruct((n, 6 * coutb, fw), _F32),
        grid=(n,),
        in_specs=[
            pl.BlockSpec((None, 8 * cina, fw), lambda i: (i, 0, 0)),
            pl.BlockSpec((9, couta, 3 * cina), lambda i: (0, 0, 0)),
            pl.BlockSpec((couta, 1), lambda i: (0, 0)),
            pl.BlockSpec((9, coutb, 3 * couta), lambda i: (0, 0, 0)),
            pl.BlockSpec((coutb, 1), lambda i: (0, 0)),
        ],
        out_specs=pl.BlockSpec((None, 6 * coutb, fw), lambda i: (i, 0, 0)),
        scratch_shapes=[pltpu.VMEM((8 * couta, fw), _F32)],
        compiler_params=pltpu.CompilerParams(
            dimension_semantics=("parallel",),
            vmem_limit_bytes=60 * 1024 * 1024),
    )(x, _w9(wa), ba.reshape(couta, 1).astype(_F32),
      _w9(wb), bb.reshape(coutb, 1).astype(_F32))


def _tower(x5, params):
    n, c, d, h, w = x5.shape
    (w0, b0), (w1, b1), (w2, b2), (w3, b3), (w4, b4), (w5, b5), (w6, b6), \
        (w7, b7) = params

    y = _stage1(x5, w0, b0, w1, b1)

    y = y.reshape(n, 6, 16, h, w)[:, :, :, :h - 4, :w - 4]
    y = y.reshape(n, 6, 16, (h - 4) // 2, 2, (w - 4) // 2, 2).max(axis=(4, 6))
    h2, w2_ = (h - 4) // 2, (w - 4) // 2
    y = y.reshape(n, 6, 16, h2 * w2_)
    y = jnp.pad(y, ((0, 0), (1, 1), (0, 0), (0, 3968 - h2 * w2_)))
    y = y.reshape(n, 128, 3968)

    y = _pair(y, w2, b2, w3, b3, cina=16, couta=16, coutb=32, fw=3968,
              wf=w2_, chunks=((0, 1280), (1280, 1280), (2560, 1280)),
              relu_b=True)

    y = y.reshape(n, 6, 32, 3968)[:, :, :, :h2 * w2_]
    y = y.reshape(n, 6, 32, h2, w2_)[:, :, :, :h2 - 4, :w2_ - 4]
    y = y.reshape(n, 6, 32, (h2 - 4) // 2, 2, (w2_ - 4) // 2, 2)
    y = y.max(axis=(4, 6))
    h3, w3_ = (h2 - 4) // 2, (w2_ - 4) // 2
    y = y.reshape(n, 6, 32, h3 * w3_)
    y = jnp.pad(y, ((0, 0), (1, 1), (0, 0), (0, 896 - h3 * w3_)))
    y = y.reshape(n, 256, 896)

    y = _pair(y, w4, b4, w5, b5, cina=32, couta=64, coutb=32, fw=896,
              wf=w3_, chunks=((0, 448), (448, 333)), relu_b=True)

    y = y.reshape(n, 6, 32, 896)[:, :, :, :h3 * w3_]
    y = y.reshape(n, 6, 32, h3, w3_)[:, :, :, :(h3 - 4) // 2 * 2,
                                     :(w3_ - 4) // 2 * 2]
    h4, w4_ = (h3 - 4) // 2, (w3_ - 4) // 2
    y = y.reshape(n, 6, 32, h4, 2, w4_, 2).max(axis=(4, 6))
    y = y.reshape(n, 6, 32, h4 * w4_)
    y = jnp.pad(y, ((0, 0), (1, 1), (0, 0), (0, 256 - h4 * w4_)))
    y = y.reshape(n, 256, 256)

    y = _pair(y, w6, b6, w7, b7, cina=32, couta=64, coutb=32, fw=256,
              wf=w4_, chunks=((0, 118),), relu_b=False)

    y = y.reshape(n, 6, 32, 256)[:, :, :, :h4 * w4_]
    y = y.reshape(n, 6, 32, h4, w4_)[:, :, :, :h4 - 4, :w4_ - 4]
    y = jnp.transpose(y, (0, 1, 3, 4, 2))
    return y.reshape(n, 6 * (h4 - 4) * (w4_ - 4), 32)


def _head_body(a_ref, t_ref, wa_ref, wt_ref, b_ref, soft_ref, out_ref):
    sa = jnp.mean(a_ref[...], axis=1)
    st = jnp.mean(t_ref[...], axis=1)
    logits = (jnp.sum(sa * wa_ref[...], axis=1, keepdims=True)
              + jnp.sum(st * wt_ref[...], axis=1, keepdims=True)
              + b_ref[...])
    out_ref[...] = logits
    soft_ref[...] = jnp.ones_like(logits)


def kernel(dwi_x, t2_x,
           dwi_w0, dwi_b0, dwi_w1, dwi_b1, dwi_w2, dwi_b2, dwi_w3, dwi_b3,
           dwi_w4, dwi_b4, dwi_w5, dwi_b5, dwi_w6, dwi_b6, dwi_w7, dwi_b7,
           t2_w0, t2_b0, t2_w1, t2_b1, t2_w2, t2_b2, t2_w3, t2_b3,
           t2_w4, t2_b4, t2_w5, t2_b5, t2_w6, t2_b6, t2_w7, t2_b7,
           lin_w, lin_b):
    p_dwi = [(dwi_w0, dwi_b0), (dwi_w1, dwi_b1), (dwi_w2, dwi_b2),
             (dwi_w3, dwi_b3), (dwi_w4, dwi_b4), (dwi_w5, dwi_b5),
             (dwi_w6, dwi_b6), (dwi_w7, dwi_b7)]
    p_t2 = [(t2_w0, t2_b0), (t2_w1, t2_b1), (t2_w2, t2_b2), (t2_w3, t2_b3),
            (t2_w4, t2_b4), (t2_w5, t2_b5), (t2_w6, t2_b6), (t2_w7, t2_b7)]
    f_dwi = _tower(dwi_x, p_dwi)
    f_t2 = _tower(t2_x, p_t2)
    n = f_dwi.shape[0]
    soft, out = pl.pallas_call(
        _head_body,
        out_shape=(jax.ShapeDtypeStruct((n, 1), _F32),
                   jax.ShapeDtypeStruct((n, 1), _F32)),
    )(f_dwi, f_t2, lin_w[:, :32], lin_w[:, 32:], lin_b.reshape(1, 1))
    return soft, out


# speedup vs baseline: 1.3059x; 1.0269x over previous
import functools

import jax
import jax.numpy as jnp
from jax.experimental import pallas as pl
from jax.experimental.pallas import tpu as pltpu

_F32 = jnp.float32
_BF16 = jnp.bfloat16


def _stage1_body(x_ref, w0_ref, b0_ref, w1_ref, b1_ref, o_ref, s0_ref, *,
                 c_in, d, hw, wf):
    m_plane = hw - 2 * wf - 2
    hw_s = hw + 128

    ch0 = 4096
    n_chunks0 = hw // ch0

    omax = 2 * wf + 2

    def l0_chunk(dd, loc, cw):
        acc = b0_ref[...] + jnp.zeros((8, cw), _F32)
        for kd in range(3):
            xw = x_ref[:, pl.ds((dd + kd) * hw + loc, cw + omax)]
            for kw in range(3):
                xr = jax.lax.slice_in_dim(xw, kw, kw + cw + 2 * wf, axis=1)
                for kh in range(3):
                    sl = jax.lax.slice_in_dim(xr, kh * wf, kh * wf + cw,
                                              axis=1)
                    acc = acc + jnp.dot(
                        w0_ref[pl.ds((kd * 9 + kh * 3 + kw) * 8, 8)], sl,
                        preferred_element_type=_F32)
        acc = jnp.maximum(acc, 0.0).astype(_F32)
        s0_ref[pl.ds(dd * 8, 8), pl.ds(loc, cw)] = acc

    def body0(k, carry):
        l0_chunk(k // n_chunks0, (k % n_chunks0) * ch0, ch0)
        return carry

    jax.lax.fori_loop(0, (d - 2) * n_chunks0 - 1, body0, 0)
    l0_chunk(d - 3, (n_chunks0 - 1) * ch0, m_plane - (n_chunks0 - 1) * ch0)

    ch1 = 1792
    n_chunks1 = 9

    def l1_pair(p, c0):
        outs = []
        for half in range(2):
            dd = 2 * p + half
            acc = b1_ref[...] + jnp.zeros((16, ch1), _F32)
            xw = s0_ref[pl.ds(dd * 8, 24), pl.ds(c0, ch1 + omax)]
            for kw in range(3):
                xr = jax.lax.slice_in_dim(xw, kw, kw + ch1 + 2 * wf, axis=1)
                for kh in range(3):
                    sl = jax.lax.slice_in_dim(xr, kh * wf, kh * wf + ch1,
                                              axis=1)
                    acc = acc + jnp.dot(
                        w1_ref[pl.ds((kh * 3 + kw) * 16, 16)], sl,
                        preferred_element_type=_F32)
            outs.append(acc)
        mx = jnp.maximum(jnp.maximum(outs[0], outs[1]), 0.0).astype(_F32)
        o_ref[pl.ds(p * 16, 16), pl.ds(c0, ch1)] = mx

    def body1(j, carry):
        l1_pair(j // n_chunks1, (j % n_chunks1) * ch1)
        return carry

    jax.lax.fori_loop(0, 6 * n_chunks1, body1, 0)


def _pair_body(x_ref, wa_ref, ba_ref, wb_ref, bb_ref, o_ref, s_ref, *,
               cina, couta, coutb, fw, wf, chunks, relu_b):
    for dd in range(6):
        for c0, cw in chunks:
            acc = ba_ref[...] + jnp.zeros((couta, cw), _F32)
            for t in range(9):
                kh, kw = divmod(t, 3)
                acc = acc + jnp.dot(
                    wa_ref[t],
                    x_ref[pl.ds(dd * cina, 3 * cina),
                          pl.ds(kh * wf + kw + c0, cw)],
                    preferred_element_type=_F32)
            acc = jnp.maximum(acc, 0.0).astype(_F32)
            s_ref[pl.ds((dd + 1) * couta, couta), pl.ds(c0, cw)] = acc
    z = jnp.zeros((couta, fw), _F32)
    s_ref[pl.ds(0, couta), :] = z
    s_ref[pl.ds(7 * couta, couta), :] = z
    for dd in range(6):
        for c0, cw in chunks:
            acc = bb_ref[...] + jnp.zeros((coutb, cw), _F32)
            for t in range(9):
                kh, kw = divmod(t, 3)
                acc = acc + jnp.dot(
                    wb_ref[t],
                    s_ref[pl.ds(dd * couta, 3 * couta),
                          pl.ds(kh * wf + kw + c0, cw)],
                    preferred_element_type=_F32)
            if relu_b:
                acc = jnp.maximum(acc, 0.0)
            o_ref[pl.ds(dd * coutb, coutb), pl.ds(c0, cw)] = acc.astype(_F32)


def _w27(w):
    return jnp.transpose(w, (2, 3, 4, 0, 1)).reshape(27, w.shape[0], -1)


def _w9(w):
    cout, cin = w.shape[0], w.shape[1]
    return jnp.transpose(w, (3, 4, 0, 2, 1)).reshape(
        9, cout, 3 * cin).astype(_F32)


def _stage1(x, w0, b0, w1, b1):
    n, c, d, h, w = x.shape
    hw = h * w
    xf = x.reshape(n, c, d * hw)
    body = functools.partial(_stage1_body, c_in=c, d=d, hw=hw, wf=w)
    return pl.pallas_call(
        body,
        out_shape=jax.ShapeDtypeStruct((n, 96, hw), _F32),
        grid=(n,),
        in_specs=[
            pl.BlockSpec((None, c, d * hw), lambda i: (i, 0, 0)),
            pl.BlockSpec((216, c), lambda i: (0, 0)),
            pl.BlockSpec((8, 1), lambda i: (0, 0)),
            pl.BlockSpec((144, 24), lambda i: (0, 0)),
            pl.BlockSpec((16, 1), lambda i: (0, 0)),
        ],
        out_specs=pl.BlockSpec((None, 96, hw), lambda i: (i, 0, 0)),
        scratch_shapes=[pltpu.VMEM((112, hw + 128), _F32)],
        compiler_params=pltpu.CompilerParams(
            dimension_semantics=("parallel",),
            vmem_limit_bytes=60 * 1024 * 1024),
    )(xf, _w27(w0).reshape(216, c), b0.reshape(8, 1),
      _w9(w1).reshape(144, 24), b1.reshape(16, 1))


def _pair(x, wa, ba, wb, bb, *, cina, couta, coutb, fw, wf, chunks, relu_b):
    n = x.shape[0]
    body = functools.partial(
        _pair_body, cina=cina, couta=couta, coutb=coutb, fw=fw, wf=wf,
        chunks=chunks, relu_b=relu_b)
    return pl.pallas_call(
        body,
        out_shape=jax.ShapeDtypeStruct((n, 6 * coutb, fw), _F32),
        grid=(n,),
        in_specs=[
            pl.BlockSpec((None, 8 * cina, fw), lambda i: (i, 0, 0)),
            pl.BlockSpec((9, couta, 3 * cina), lambda i: (0, 0, 0)),
            pl.BlockSpec((couta, 1), lambda i: (0, 0)),
            pl.BlockSpec((9, coutb, 3 * couta), lambda i: (0, 0, 0)),
            pl.BlockSpec((coutb, 1), lambda i: (0, 0)),
        ],
        out_specs=pl.BlockSpec((None, 6 * coutb, fw), lambda i: (i, 0, 0)),
        scratch_shapes=[pltpu.VMEM((8 * couta, fw), _F32)],
        compiler_params=pltpu.CompilerParams(
            dimension_semantics=("parallel",),
            vmem_limit_bytes=60 * 1024 * 1024),
    )(x, _w9(wa), ba.reshape(couta, 1).astype(_F32),
      _w9(wb), bb.reshape(coutb, 1).astype(_F32))


def _tower(x5, params):
    n, c, d, h, w = x5.shape
    (w0, b0), (w1, b1), (w2, b2), (w3, b3), (w4, b4), (w5, b5), (w6, b6), \
        (w7, b7) = params

    y = _stage1(x5, w0, b0, w1, b1)

    y = y.reshape(n, 6, 16, h, w)[:, :, :, :h - 4, :w - 4]
    y = y.reshape(n, 6, 16, (h - 4) // 2, 2, (w - 4) // 2, 2).max(axis=(4, 6))
    h2, w2_ = (h - 4) // 2, (w - 4) // 2
    y = y.reshape(n, 6, 16, h2 * w2_)
    y = jnp.pad(y, ((0, 0), (1, 1), (0, 0), (0, 3968 - h2 * w2_)))
    y = y.reshape(n, 128, 3968)

    y = _pair(y, w2, b2, w3, b3, cina=16, couta=16, coutb=32, fw=3968,
              wf=w2_, chunks=((0, 1280), (1280, 1280), (2560, 1280)),
              relu_b=True)

    y = y.reshape(n, 6, 32, 3968)[:, :, :, :h2 * w2_]
    y = y.reshape(n, 6, 32, h2, w2_)[:, :, :, :h2 - 4, :w2_ - 4]
    y = y.reshape(n, 6, 32, (h2 - 4) // 2, 2, (w2_ - 4) // 2, 2)
    y = y.max(axis=(4, 6))
    h3, w3_ = (h2 - 4) // 2, (w2_ - 4) // 2
    y = y.reshape(n, 6, 32, h3 * w3_)
    y = jnp.pad(y, ((0, 0), (1, 1), (0, 0), (0, 896 - h3 * w3_)))
    y = y.reshape(n, 256, 896)

    y = _pair(y, w4, b4, w5, b5, cina=32, couta=64, coutb=32, fw=896,
              wf=w3_, chunks=((0, 448), (448, 333)), relu_b=True)

    y = y.reshape(n, 6, 32, 896)[:, :, :, :h3 * w3_]
    y = y.reshape(n, 6, 32, h3, w3_)[:, :, :, :(h3 - 4) // 2 * 2,
                                     :(w3_ - 4) // 2 * 2]
    h4, w4_ = (h3 - 4) // 2, (w3_ - 4) // 2
    y = y.reshape(n, 6, 32, h4, 2, w4_, 2).max(axis=(4, 6))
    y = y.reshape(n, 6, 32, h4 * w4_)
    y = jnp.pad(y, ((0, 0), (1, 1), (0, 0), (0, 256 - h4 * w4_)))
    y = y.reshape(n, 256, 256)

    y = _pair(y, w6, b6, w7, b7, cina=32, couta=64, coutb=32, fw=256,
              wf=w4_, chunks=((0, 118),), relu_b=False)

    y = y.reshape(n, 6, 32, 256)[:, :, :, :h4 * w4_]
    y = y.reshape(n, 6, 32, h4, w4_)[:, :, :, :h4 - 4, :w4_ - 4]
    y = jnp.transpose(y, (0, 1, 3, 4, 2))
    return y.reshape(n, 6 * (h4 - 4) * (w4_ - 4), 32)


def _head_body(a_ref, t_ref, wa_ref, wt_ref, b_ref, soft_ref, out_ref):
    sa = jnp.mean(a_ref[...], axis=1)
    st = jnp.mean(t_ref[...], axis=1)
    logits = (jnp.sum(sa * wa_ref[...], axis=1, keepdims=True)
              + jnp.sum(st * wt_ref[...], axis=1, keepdims=True)
              + b_ref[...])
    out_ref[...] = logits
    soft_ref[...] = jnp.ones_like(logits)


def kernel(dwi_x, t2_x,
           dwi_w0, dwi_b0, dwi_w1, dwi_b1, dwi_w2, dwi_b2, dwi_w3, dwi_b3,
           dwi_w4, dwi_b4, dwi_w5, dwi_b5, dwi_w6, dwi_b6, dwi_w7, dwi_b7,
           t2_w0, t2_b0, t2_w1, t2_b1, t2_w2, t2_b2, t2_w3, t2_b3,
           t2_w4, t2_b4, t2_w5, t2_b5, t2_w6, t2_b6, t2_w7, t2_b7,
           lin_w, lin_b):
    p_dwi = [(dwi_w0, dwi_b0), (dwi_w1, dwi_b1), (dwi_w2, dwi_b2),
             (dwi_w3, dwi_b3), (dwi_w4, dwi_b4), (dwi_w5, dwi_b5),
             (dwi_w6, dwi_b6), (dwi_w7, dwi_b7)]
    p_t2 = [(t2_w0, t2_b0), (t2_w1, t2_b1), (t2_w2, t2_b2), (t2_w3, t2_b3),
            (t2_w4, t2_b4), (t2_w5, t2_b5), (t2_w6, t2_b6), (t2_w7, t2_b7)]
    f_dwi = _tower(dwi_x, p_dwi)
    f_t2 = _tower(t2_x, p_t2)
    n = f_dwi.shape[0]
    soft, out = pl.pallas_call(
        _head_body,
        out_shape=(jax.ShapeDtypeStruct((n, 1), _F32),
                   jax.ShapeDtypeStruct((n, 1), _F32)),
    )(f_dwi, f_t2, lin_w[:, :32], lin_w[:, 32:], lin_b.reshape(1, 1))
    return soft, out


# speedup vs baseline: 1.3170x; 1.0085x over previous
import functools

import jax
import jax.numpy as jnp
from jax.experimental import pallas as pl
from jax.experimental.pallas import tpu as pltpu

_F32 = jnp.float32
_BF16 = jnp.bfloat16


def _stage1_body(x_ref, w0_ref, b0_ref, w1_ref, b1_ref, o_ref, s0_ref, *,
                 c_in, d, hw, wf):
    m_plane = hw - 2 * wf - 2
    hw_s = hw + 128

    ch0 = 4096
    n_chunks0 = hw // ch0

    omax = 2 * wf + 2

    def l0_chunk(dd, loc, cw):
        acc = b0_ref[...] + jnp.zeros((8, cw), _F32)
        for kd in range(3):
            xw = x_ref[:, pl.ds((dd + kd) * hw + loc, cw + omax)]
            for kw in range(3):
                xr = jax.lax.slice_in_dim(xw, kw, kw + cw + 2 * wf, axis=1)
                for kh in range(3):
                    sl = jax.lax.slice_in_dim(xr, kh * wf, kh * wf + cw,
                                              axis=1)
                    acc = acc + jnp.dot(
                        w0_ref[pl.ds((kd * 9 + kh * 3 + kw) * 8, 8)], sl,
                        preferred_element_type=_F32)
        acc = jnp.maximum(acc, 0.0).astype(_F32)
        s0_ref[pl.ds(dd * 8, 8), pl.ds(loc, cw)] = acc

    def body0(k, carry):
        l0_chunk(k // n_chunks0, (k % n_chunks0) * ch0, ch0)
        return carry

    jax.lax.fori_loop(0, (d - 2) * n_chunks0 - 1, body0, 0)
    l0_chunk(d - 3, (n_chunks0 - 1) * ch0, m_plane - (n_chunks0 - 1) * ch0)

    ch1 = 1792
    n_chunks1 = 9

    def l1_pair(p, c0):
        xw = s0_ref[pl.ds(p * 16, 32), pl.ds(c0, ch1 + omax)]
        rots = [jax.lax.slice_in_dim(xw, kw, kw + ch1 + 2 * wf, axis=1)
                for kw in range(3)]
        outs = []
        for half in range(2):
            acc = b1_ref[...] + jnp.zeros((16, ch1), _F32)
            for kw in range(3):
                for kh in range(3):
                    sl = jax.lax.slice(
                        rots[kw], (half * 8, kh * wf),
                        (half * 8 + 24, kh * wf + ch1))
                    acc = acc + jnp.dot(
                        w1_ref[pl.ds((kh * 3 + kw) * 16, 16)], sl,
                        preferred_element_type=_F32)
            outs.append(acc)
        mx = jnp.maximum(jnp.maximum(outs[0], outs[1]), 0.0).astype(_F32)
        o_ref[pl.ds(p * 16, 16), pl.ds(c0, ch1)] = mx

    def body1(j, carry):
        l1_pair(j // n_chunks1, (j % n_chunks1) * ch1)
        return carry

    jax.lax.fori_loop(0, 6 * n_chunks1, body1, 0)


def _pair_body(x_ref, wa_ref, ba_ref, wb_ref, bb_ref, o_ref, s_ref, *,
               cina, couta, coutb, fw, wf, chunks, relu_b):
    for dd in range(6):
        for c0, cw in chunks:
            acc = ba_ref[...] + jnp.zeros((couta, cw), _F32)
            for t in range(9):
                kh, kw = divmod(t, 3)
                acc = acc + jnp.dot(
                    wa_ref[t],
                    x_ref[pl.ds(dd * cina, 3 * cina),
                          pl.ds(kh * wf + kw + c0, cw)],
                    preferred_element_type=_F32)
            acc = jnp.maximum(acc, 0.0).astype(_F32)
            s_ref[pl.ds((dd + 1) * couta, couta), pl.ds(c0, cw)] = acc
    z = jnp.zeros((couta, fw), _F32)
    s_ref[pl.ds(0, couta), :] = z
    s_ref[pl.ds(7 * couta, couta), :] = z
    for dd in range(6):
        for c0, cw in chunks:
            acc = bb_ref[...] + jnp.zeros((coutb, cw), _F32)
            for t in range(9):
                kh, kw = divmod(t, 3)
                acc = acc + jnp.dot(
                    wb_ref[t],
                    s_ref[pl.ds(dd * couta, 3 * couta),
                          pl.ds(kh * wf + kw + c0, cw)],
                    preferred_element_type=_F32)
            if relu_b:
                acc = jnp.maximum(acc, 0.0)
            o_ref[pl.ds(dd * coutb, coutb), pl.ds(c0, cw)] = acc.astype(_F32)


def _w27(w):
    return jnp.transpose(w, (2, 3, 4, 0, 1)).reshape(27, w.shape[0], -1)


def _w9(w):
    cout, cin = w.shape[0], w.shape[1]
    return jnp.transpose(w, (3, 4, 0, 2, 1)).reshape(
        9, cout, 3 * cin).astype(_F32)


def _stage1(x, w0, b0, w1, b1):
    n, c, d, h, w = x.shape
    hw = h * w
    xf = x.reshape(n, c, d * hw)
    body = functools.partial(_stage1_body, c_in=c, d=d, hw=hw, wf=w)
    return pl.pallas_call(
        body,
        out_shape=jax.ShapeDtypeStruct((n, 96, hw), _F32),
        grid=(n,),
        in_specs=[
            pl.BlockSpec((None, c, d * hw), lambda i: (i, 0, 0)),
            pl.BlockSpec((216, c), lambda i: (0, 0)),
            pl.BlockSpec((8, 1), lambda i: (0, 0)),
            pl.BlockSpec((144, 24), lambda i: (0, 0)),
            pl.BlockSpec((16, 1), lambda i: (0, 0)),
        ],
        out_specs=pl.BlockSpec((None, 96, hw), lambda i: (i, 0, 0)),
        scratch_shapes=[pltpu.VMEM((112, hw + 128), _F32)],
        compiler_params=pltpu.CompilerParams(
            dimension_semantics=("parallel",),
            vmem_limit_bytes=60 * 1024 * 1024),
    )(xf, _w27(w0).reshape(216, c), b0.reshape(8, 1),
      _w9(w1).reshape(144, 24), b1.reshape(16, 1))


def _pair(x, wa, ba, wb, bb, *, cina, couta, coutb, fw, wf, chunks, relu_b):
    n = x.shape[0]
    body = functools.partial(
        _pair_body, cina=cina, couta=couta, coutb=coutb, fw=fw, wf=wf,
        chunks=chunks, relu_b=relu_b)
    return pl.pallas_call(
        body,
        out_shape=jax.ShapeDtypeStruct((n, 6 * coutb, fw), _F32),
        grid=(n,),
        in_specs=[
            pl.BlockSpec((None, 8 * cina, fw), lambda i: (i, 0, 0)),
            pl.BlockSpec((9, couta, 3 * cina), lambda i: (0, 0, 0)),
            pl.BlockSpec((couta, 1), lambda i: (0, 0)),
            pl.BlockSpec((9, coutb, 3 * couta), lambda i: (0, 0, 0)),
            pl.BlockSpec((coutb, 1), lambda i: (0, 0)),
        ],
        out_specs=pl.BlockSpec((None, 6 * coutb, fw), lambda i: (i, 0, 0)),
        scratch_shapes=[pltpu.VMEM((8 * couta, fw), _F32)],
        compiler_params=pltpu.CompilerParams(
            dimension_semantics=("parallel",),
            vmem_limit_bytes=60 * 1024 * 1024),
    )(x, _w9(wa), ba.reshape(couta, 1).astype(_F32),
      _w9(wb), bb.reshape(coutb, 1).astype(_F32))


def _tower(x5, params):
    n, c, d, h, w = x5.shape
    (w0, b0), (w1, b1), (w2, b2), (w3, b3), (w4, b4), (w5, b5), (w6, b6), \
        (w7, b7) = params

    y = _stage1(x5, w0, b0, w1, b1)

    y = y.reshape(n, 6, 16, h, w)[:, :, :, :h - 4, :w - 4]
    y = y.reshape(n, 6, 16, (h - 4) // 2, 2, (w - 4) // 2, 2).max(axis=(4, 6))
    h2, w2_ = (h - 4) // 2, (w - 4) // 2
    y = y.reshape(n, 6, 16, h2 * w2_)
    y = jnp.pad(y, ((0, 0), (1, 1), (0, 0), (0, 3968 - h2 * w2_)))
    y = y.reshape(n, 128, 3968)

    y = _pair(y, w2, b2, w3, b3, cina=16, couta=16, coutb=32, fw=3968,
              wf=w2_, chunks=((0, 1280), (1280, 1280), (2560, 1280)),
              relu_b=True)

    y = y.reshape(n, 6, 32, 3968)[:, :, :, :h2 * w2_]
    y = y.reshape(n, 6, 32, h2, w2_)[:, :, :, :h2 - 4, :w2_ - 4]
    y = y.reshape(n, 6, 32, (h2 - 4) // 2, 2, (w2_ - 4) // 2, 2)
    y = y.max(axis=(4, 6))
    h3, w3_ = (h2 - 4) // 2, (w2_ - 4) // 2
    y = y.reshape(n, 6, 32, h3 * w3_)
    y = jnp.pad(y, ((0, 0), (1, 1), (0, 0), (0, 896 - h3 * w3_)))
    y = y.reshape(n, 256, 896)

    y = _pair(y, w4, b4, w5, b5, cina=32, couta=64, coutb=32, fw=896,
              wf=w3_, chunks=((0, 448), (448, 333)), relu_b=True)

    y = y.reshape(n, 6, 32, 896)[:, :, :, :h3 * w3_]
    y = y.reshape(n, 6, 32, h3, w3_)[:, :, :, :(h3 - 4) // 2 * 2,
                                     :(w3_ - 4) // 2 * 2]
    h4, w4_ = (h3 - 4) // 2, (w3_ - 4) // 2
    y = y.reshape(n, 6, 32, h4, 2, w4_, 2).max(axis=(4, 6))
    y = y.reshape(n, 6, 32, h4 * w4_)
    y = jnp.pad(y, ((0, 0), (1, 1), (0, 0), (0, 256 - h4 * w4_)))
    y = y.reshape(n, 256, 256)

    y = _pair(y, w6, b6, w7, b7, cina=32, couta=64, coutb=32, fw=256,
              wf=w4_, chunks=((0, 118),), relu_b=False)

    y = y.reshape(n, 6, 32, 256)[:, :, :, :h4 * w4_]
    y = y.reshape(n, 6, 32, h4, w4_)[:, :, :, :h4 - 4, :w4_ - 4]
    y = jnp.transpose(y, (0, 1, 3, 4, 2))
    return y.reshape(n, 6 * (h4 - 4) * (w4_ - 4), 32)


def _head_body(a_ref, t_ref, wa_ref, wt_ref, b_ref, soft_ref, out_ref):
    sa = jnp.mean(a_ref[...], axis=1)
    st = jnp.mean(t_ref[...], axis=1)
    logits = (jnp.sum(sa * wa_ref[...], axis=1, keepdims=True)
              + jnp.sum(st * wt_ref[...], axis=1, keepdims=True)
              + b_ref[...])
    out_ref[...] = logits
    soft_ref[...] = jnp.ones_like(logits)


def kernel(dwi_x, t2_x,
           dwi_w0, dwi_b0, dwi_w1, dwi_b1, dwi_w2, dwi_b2, dwi_w3, dwi_b3,
           dwi_w4, dwi_b4, dwi_w5, dwi_b5, dwi_w6, dwi_b6, dwi_w7, dwi_b7,
           t2_w0, t2_b0, t2_w1, t2_b1, t2_w2, t2_b2, t2_w3, t2_b3,
           t2_w4, t2_b4, t2_w5, t2_b5, t2_w6, t2_b6, t2_w7, t2_b7,
           lin_w, lin_b):
    p_dwi = [(dwi_w0, dwi_b0), (dwi_w1, dwi_b1), (dwi_w2, dwi_b2),
             (dwi_w3, dwi_b3), (dwi_w4, dwi_b4), (dwi_w5, dwi_b5),
             (dwi_w6, dwi_b6), (dwi_w7, dwi_b7)]
    p_t2 = [(t2_w0, t2_b0), (t2_w1, t2_b1), (t2_w2, t2_b2), (t2_w3, t2_b3),
            (t2_w4, t2_b4), (t2_w5, t2_b5), (t2_w6, t2_b6), (t2_w7, t2_b7)]
    f_dwi = _tower(dwi_x, p_dwi)
    f_t2 = _tower(t2_x, p_t2)
    n = f_dwi.shape[0]
    soft, out = pl.pallas_call(
        _head_body,
        out_shape=(jax.ShapeDtypeStruct((n, 1), _F32),
                   jax.ShapeDtypeStruct((n, 1), _F32)),
    )(f_dwi, f_t2, lin_w[:, :32], lin_w[:, 32:], lin_b.reshape(1, 1))
    return soft, out
